# Initial kernel scaffold; baseline (speedup 1.0000x reference)
import jax
import jax.numpy as jnp
from jax.experimental import pallas as pl


def kernel(hidden_states, gate_w, expert_w):
    raise NotImplementedError("write your pallas kernel here")


# speedup vs baseline: 1.4601x; 1.4601x over previous
import functools

import jax
import jax.numpy as jnp
from jax.experimental import pallas as pl
from jax.experimental.pallas import tpu as pltpu

_HIDDEN = 2048
_EXPERTS = 8
_BT = 512


def _moe_block_kernel(x_ref, gate_ref, w_ref, out_ref, logits_ref, dw_ref):
    n = pl.program_id(1)
    e = pl.program_id(2)

    @pl.when((e == 0) & (n == 0))
    def _router():
        x = x_ref[...]
        logits = jax.lax.dot_general(
            x, gate_ref[...], (((1,), (1,)), ((), ())),
            preferred_element_type=jnp.float32,
        )
        logits_ref[...] = logits
        ids = jax.lax.broadcasted_iota(jnp.int32, logits.shape, 1)
        m1 = jnp.max(logits, axis=1, keepdims=True)
        e1 = jnp.min(jnp.where(logits == m1, ids, _EXPERTS), axis=1, keepdims=True)
        mask1 = ids == e1
        l2 = jnp.where(mask1, -jnp.inf, logits)
        m2 = jnp.max(l2, axis=1, keepdims=True)
        e2 = jnp.min(jnp.where(l2 == m2, ids, _EXPERTS), axis=1, keepdims=True)
        mask2 = ids == e2
        r = jnp.exp(m2 - m1)
        w1 = 1.0 / (1.0 + r)
        w2 = 1.0 - w1
        dw_ref[...] = jnp.where(mask1, w1, 0.0) + jnp.where(mask2, w2, 0.0)

    x = x_ref[...]
    w = w_ref[0]
    y = jax.lax.dot_general(
        x, w, (((1,), (1,)), ((), ())),
        preferred_element_type=jnp.float32,
    )
    dw = dw_ref[...]
    ids = jax.lax.broadcasted_iota(jnp.int32, dw.shape, 1)
    coeff = jnp.sum(jnp.where(ids == e, dw, 0.0), axis=1, keepdims=True)
    prev = jnp.where(e == 0, 0.0, out_ref[...])
    out_ref[...] = prev + coeff * y


def kernel(hidden_states, gate_w, expert_w):
    t, h = hidden_states.shape
    n_tb = t // _BT
    bn = h // 2
    out, logits = pl.pallas_call(
        _moe_block_kernel,
        grid=(n_tb, 2, _EXPERTS),
        in_specs=[
            pl.BlockSpec((_BT, h), lambda i, n, e: (i, 0)),
            pl.BlockSpec((_EXPERTS, h), lambda i, n, e: (0, 0)),
            pl.BlockSpec((1, bn, h), lambda i, n, e: (e, n, 0)),
        ],
        out_specs=[
            pl.BlockSpec((_BT, bn), lambda i, n, e: (i, n)),
            pl.BlockSpec((_BT, _EXPERTS), lambda i, n, e: (i, 0)),
        ],
        out_shape=[
            jax.ShapeDtypeStruct((t, h), jnp.float32),
            jax.ShapeDtypeStruct((t, _EXPERTS), jnp.float32),
        ],
        scratch_shapes=[pltpu.VMEM((_BT, _EXPERTS), jnp.float32)],
        compiler_params=pltpu.CompilerParams(
            dimension_semantics=("arbitrary", "arbitrary", "arbitrary"),
        ),
    )(hidden_states, gate_w, expert_w)
    return out, logits
